# baseline (device time: 24838 ns/iter reference)
import os

import jax
import jax.numpy as jnp
from jax import lax
from jax.experimental import pallas as pl
from jax.experimental.pallas import tpu as pltpu

N_DEV = 4
VC = 2048
_NO_COMM = os.path.exists(os.path.join(os.path.dirname(__file__), "NO_COMM"))


def kernel(x, W, labels):
    T, D = x.shape
    _, V_shard = W.shape
    n_chunks = V_shard // VC

    def body(x_ref, w_ref, lab_ref, out_ref,
             xb_ref, stat_ref, gather_ref, send_sems, recv_sems):
        my_pos = lax.axis_index("i")
        j = pl.program_id(0)
        barrier_sem = None if _NO_COMM else pltpu.get_barrier_semaphore()

        if not _NO_COMM:
            @pl.when(j == 0)
            def _():
                for o in range(1, N_DEV):
                    peer = lax.rem(my_pos + o, N_DEV)
                    pl.semaphore_signal(barrier_sem, inc=1, device_id=(peer,),
                                        device_id_type=pl.DeviceIdType.MESH)

        @pl.when(j == 0)
        def _():
            xb_ref[:, :] = x_ref[:, :].astype(jnp.bfloat16)

        wb = w_ref[:, :].astype(jnp.bfloat16)
        logits = lax.dot_general(
            xb_ref[:, :], wb, (((1,), (0,)), ((), ())),
            preferred_element_type=jnp.float32,
        )

        s_j = jnp.sum(jnp.exp(logits), axis=1)
        lab_local = lab_ref[:] - my_pos * V_shard - j * VC
        col = lax.broadcasted_iota(jnp.int32, (T, VC), 1)
        l_j = jnp.sum(jnp.where(col == lab_local[:, None], logits, 0.0),
                      axis=1)

        @pl.when(j == 0)
        def _():
            stat_ref[0, :] = s_j
            stat_ref[1, :] = l_j

        @pl.when(j > 0)
        def _():
            stat_ref[0, :] = stat_ref[0, :] + s_j
            stat_ref[1, :] = stat_ref[1, :] + l_j

        @pl.when(j == n_chunks - 1)
        def _():
            if _NO_COMM:
                out_ref[:] = jnp.log(stat_ref[0, :]) - stat_ref[1, :]
                return
            pl.semaphore_wait(barrier_sem, N_DEV - 1)
            rdmas = []
            for o in range(1, N_DEV):
                peer = lax.rem(my_pos + o, N_DEV)
                rdma = pltpu.make_async_remote_copy(
                    src_ref=stat_ref,
                    dst_ref=gather_ref.at[o - 1],
                    send_sem=send_sems.at[o - 1],
                    recv_sem=recv_sems.at[o - 1],
                    device_id=(peer,),
                    device_id_type=pl.DeviceIdType.MESH,
                )
                rdma.start()
                rdmas.append(rdma)
            for rdma in rdmas:
                rdma.wait()

            S = stat_ref[0, :]
            L = stat_ref[1, :]
            for k in range(N_DEV - 1):
                S = S + gather_ref[k, 0, :]
                L = L + gather_ref[k, 1, :]
            out_ref[:] = jnp.log(S) - L

    return pl.pallas_call(
        body,
        grid=(n_chunks,),
        out_shape=jax.ShapeDtypeStruct((T,), jnp.float32),
        in_specs=[
            pl.BlockSpec((T, D), lambda j: (0, 0)),
            pl.BlockSpec((D, VC), lambda j: (0, j)),
            pl.BlockSpec((T,), lambda j: (0,)),
        ],
        out_specs=pl.BlockSpec((T,), lambda j: (0,)),
        scratch_shapes=[
            pltpu.VMEM((T, D), jnp.bfloat16),
            pltpu.VMEM((2, T), jnp.float32),
            pltpu.VMEM((N_DEV - 1, 2, T), jnp.float32),
            pltpu.SemaphoreType.DMA((N_DEV - 1,)),
            pltpu.SemaphoreType.DMA((N_DEV - 1,)),
        ],
        compiler_params=pltpu.CompilerParams(
            collective_id=None if _NO_COMM else 0,
            vmem_limit_bytes=100 * 1024 * 1024,
            dimension_semantics=("arbitrary",),
        ),
    )(x, W, labels)
